# baseline (speedup 1.0000x reference)
import functools

import numpy as np
import jax
import jax.numpy as jnp
from jax.experimental import pallas as pl
from jax.experimental.pallas import tpu as pltpu

_N = 10000
_E = 160000
_EMB = 300
_PROJ = 100
_G = 256
_L = 5
_TEMP = 0.04

_D = 304
_H = 608
_DP = 128
_B = 1000
_NB = _N // _B

_mask_half = np.asarray(jax.random.bernoulli(jax.random.key(42), 0.1, (_N // 2,)))
_node_mask_np = np.concatenate([_mask_half, _mask_half])
_SAMPLE_IDX = np.nonzero(_node_mask_np)[0].astype(np.int32)
_NS = int(_SAMPLE_IDX.shape[0])
_NS_PAD = ((_NS + 7) // 8) * 8


def _colmap(n):
    b = n // 2
    cm = np.empty((n, n - 1), dtype=np.int32)
    cols = np.arange(n)
    for i in range(n):
        p = i + b if i < b else i - b
        cm[i, 0] = p
        cm[i, 1:] = cols[(cols != i) & (cols != p)]
    return cm


_COLMAP_NODE = _colmap(_NS)
_COLMAP_GRAPH = _colmap(_G)

_T_I0 = np.array([0, 0, 0, 1, 1, 1, 2, 2, 2, 4], dtype=np.int32)
_T_I1 = np.array([0, 1, 2, 0, 1, 2, 0, 1, 2, 0], dtype=np.int32)


def _padw(a, rows, cols):
    return jnp.pad(a, ((0, rows - a.shape[0]), (0, cols - a.shape[1])))


def _h0_body(x_ref, a1_ref, a2_ref, o_ref):
    xv = x_ref[...]
    oh1 = (xv[:, 0:1] == jax.lax.broadcasted_iota(jnp.int32, (_B, 128), 1)
           ).astype(jnp.float32)
    oh2 = (xv[:, 1:2] == jax.lax.broadcasted_iota(jnp.int32, (_B, 8), 1)
           ).astype(jnp.float32)
    o_ref[...] = (jnp.dot(oh1, a1_ref[...], preferred_element_type=jnp.float32, precision=jax.lax.Precision.HIGHEST)
                  + jnp.dot(oh2, a2_ref[...], preferred_element_type=jnp.float32, precision=jax.lax.Precision.HIGHEST))


def _h0(x, a1p, a2p):
    return pl.pallas_call(
        _h0_body,
        grid=(_NB,),
        in_specs=[
            pl.BlockSpec((_B, 2), lambda i: (i, 0)),
            pl.BlockSpec((128, _D), lambda i: (0, 0)),
            pl.BlockSpec((8, _D), lambda i: (0, 0)),
        ],
        out_specs=pl.BlockSpec((_B, _D), lambda i: (i, 0)),
        out_shape=jax.ShapeDtypeStruct((_N, _D), jnp.float32),
    )(x, a1p, a2p)


def _mlp_body(hagg_ref, W1_ref, b1_ref, W2_ref, b2_ref, out_ref):
    agg = hagg_ref[...]
    hid = jnp.maximum(
        jnp.dot(agg.astype(jnp.bfloat16), W1_ref[...].astype(jnp.bfloat16),
                preferred_element_type=jnp.float32) + b1_ref[...],
        0.0)
    out_ref[...] = jnp.dot(hid.astype(jnp.bfloat16), W2_ref[...].astype(jnp.bfloat16),
                           preferred_element_type=jnp.float32) + b2_ref[...]


def _mlp(hagg, W1p, b1p, W2p, b2p):
    return pl.pallas_call(
        _mlp_body,
        grid=(_NB,),
        in_specs=[
            pl.BlockSpec((_B, _D), lambda i: (i, 0)),
            pl.BlockSpec((_D, _H), lambda i: (0, 0)),
            pl.BlockSpec((1, _H), lambda i: (0, 0)),
            pl.BlockSpec((_H, _D), lambda i: (0, 0)),
            pl.BlockSpec((1, _D), lambda i: (0, 0)),
        ],
        out_specs=pl.BlockSpec((_B, _D), lambda i: (i, 0)),
        out_shape=jax.ShapeDtypeStruct((_N, _D), jnp.float32),
    )(hagg, W1p, b1p, W2p, b2p)


def _bn_body(relu, out_ref, mu_ref, var_ref, sc_ref, bi_ref, y_ref):
    y = (out_ref[...] - mu_ref[...]) / jnp.sqrt(var_ref[...] + 1e-5) \
        * sc_ref[...] + bi_ref[...]
    if relu:
        y = jnp.maximum(y, 0.0)
    y_ref[...] = y


def _bn(out, mu, var, scp, bip, relu):
    return pl.pallas_call(
        functools.partial(_bn_body, relu),
        grid=(_NB,),
        in_specs=[
            pl.BlockSpec((_B, _D), lambda i: (i, 0)),
            pl.BlockSpec((1, _D), lambda i: (0, 0)),
            pl.BlockSpec((1, _D), lambda i: (0, 0)),
            pl.BlockSpec((1, _D), lambda i: (0, 0)),
            pl.BlockSpec((1, _D), lambda i: (0, 0)),
        ],
        out_specs=pl.BlockSpec((_B, _D), lambda i: (i, 0)),
        out_shape=jax.ShapeDtypeStruct((_N, _D), jnp.float32),
    )(out, mu, var, scp, bip)


def _final_body(h_ref, pw_ref, pb_ref, sidx_ref, batch_ref,
                simn_ref, simg_ref, samp_ref, seg_ref, cntg_ref):
    i = pl.program_id(0)
    f = jnp.dot(h_ref[...].astype(jnp.bfloat16), pw_ref[...].astype(jnp.bfloat16),
                preferred_element_type=jnp.float32) + pb_ref[...]
    nrm = jnp.sqrt(jnp.sum(f * f, axis=1, keepdims=True))
    fn = f / jnp.maximum(nrm, 1e-12)

    @pl.when(i == 0)
    def _():
        samp_ref[...] = jnp.zeros_like(samp_ref)
        seg_ref[...] = jnp.zeros_like(seg_ref)
        cntg_ref[...] = jnp.zeros_like(cntg_ref)

    colg = jax.lax.broadcasted_iota(jnp.int32, (1, _B), 1) + i * _B
    sel = (sidx_ref[...] == colg).astype(jnp.float32)
    samp_ref[...] += jnp.dot(sel, fn, preferred_element_type=jnp.float32, precision=jax.lax.Precision.HIGHEST)
    ohg = (jax.lax.broadcasted_iota(jnp.int32, (_G, 1), 0)
           == batch_ref[...].reshape(1, _B)).astype(jnp.float32)
    seg_ref[...] += jnp.dot(ohg, fn, preferred_element_type=jnp.float32, precision=jax.lax.Precision.HIGHEST)
    cntg_ref[...] += jnp.sum(ohg, axis=1, keepdims=True)

    @pl.when(i == pl.num_programs(0) - 1)
    def _():
        s = samp_ref[...].astype(jnp.bfloat16)
        simn_ref[...] = jax.lax.dot_general(
            s, s, (((1,), (1,)), ((), ())),
            preferred_element_type=jnp.float32) * (1.0 / _TEMP)
        m = seg_ref[...] / jnp.maximum(cntg_ref[...], 1.0)
        nr = jnp.sqrt(jnp.sum(m * m, axis=1, keepdims=True))
        fg = (m / jnp.maximum(nr, 1e-12)).astype(jnp.bfloat16)
        simg_ref[...] = jax.lax.dot_general(
            fg, fg, (((1,), (1,)), ((), ())),
            preferred_element_type=jnp.float32) * (1.0 / _TEMP)


def _final(h5, pwp, pbp, sidx, batch2d):
    return pl.pallas_call(
        _final_body,
        grid=(_NB,),
        in_specs=[
            pl.BlockSpec((_B, _D), lambda i: (i, 0)),
            pl.BlockSpec((_D, _DP), lambda i: (0, 0)),
            pl.BlockSpec((1, _DP), lambda i: (0, 0)),
            pl.BlockSpec((_NS_PAD, 1), lambda i: (0, 0)),
            pl.BlockSpec((_B, 1), lambda i: (i, 0)),
        ],
        out_specs=[
            pl.BlockSpec((_NS_PAD, _NS_PAD), lambda i: (0, 0)),
            pl.BlockSpec((_G, _G), lambda i: (0, 0)),
        ],
        out_shape=[
            jax.ShapeDtypeStruct((_NS_PAD, _NS_PAD), jnp.float32),
            jax.ShapeDtypeStruct((_G, _G), jnp.float32),
        ],
        scratch_shapes=[
            pltpu.VMEM((_NS_PAD, _DP), jnp.float32),
            pltpu.VMEM((_G, _DP), jnp.float32),
            pltpu.VMEM((_G, 1), jnp.float32),
        ],
    )(h5, pwp, pbp, sidx, batch2d)


def kernel(x, edge_index, edge_attr, batch, atom_emb1, atom_emb2, edge_emb1,
           edge_emb2, W1, b1, W2, b2, bn_scale, bn_bias, proj_W, proj_b):
    f32 = jnp.float32
    sl = jnp.arange(_N, dtype=edge_index.dtype)
    src = jnp.concatenate([edge_index[0], sl])
    dst = jnp.concatenate([edge_index[1], sl])
    c9 = jnp.concatenate([edge_attr[:, 0] * 3 + edge_attr[:, 1],
                          jnp.full((_N,), 9, edge_attr.dtype)])

    a1p = _padw(atom_emb1, 128, _D)
    a2p = _padw(atom_emb2, 8, _D)
    T = jnp.pad(edge_emb1[:, _T_I0, :] + edge_emb2[:, _T_I1, :],
                ((0, 0), (0, 6), (0, _D - _EMB)))
    W1p = jnp.pad(W1, ((0, 0), (0, _D - _EMB), (0, _H - 2 * _EMB)))
    b1p = jnp.pad(b1, ((0, 0), (0, _H - 2 * _EMB)))[:, None, :]
    W2p = jnp.pad(W2, ((0, 0), (0, _H - 2 * _EMB), (0, _D - _EMB)))
    b2p = jnp.pad(b2, ((0, 0), (0, _D - _EMB)))[:, None, :]
    scp = jnp.pad(bn_scale, ((0, 0), (0, _D - _EMB)))[:, None, :]
    bip = jnp.pad(bn_bias, ((0, 0), (0, _D - _EMB)))[:, None, :]
    pwp = _padw(proj_W, _D, _DP)
    pbp = jnp.pad(proj_b, (0, _DP - _PROJ))[None, :]

    h = _h0(x, a1p, a2p)

    for l in range(_L):
        msg = h[src] + T[l][c9]
        hagg = jax.ops.segment_sum(msg, dst, num_segments=_N)
        out = _mlp(hagg, W1p[l], b1p[l], W2p[l], b2p[l])
        mu = out.mean(axis=0)[None, :]
        var = out.var(axis=0)[None, :]
        h = _bn(out, mu, var, scp[l], bip[l], l < _L - 1)

    sidx = jnp.asarray(
        np.pad(_SAMPLE_IDX, (0, _NS_PAD - _NS), constant_values=-1))[:, None]
    sim_n, sim_g = _final(h, pwp, pbp, sidx, batch[:, None])

    logits_node = jnp.take_along_axis(sim_n[:_NS, :_NS],
                                      jnp.asarray(_COLMAP_NODE), axis=1)
    labels_node = jnp.zeros((_NS,), jnp.int32)
    logits_graph = jnp.take_along_axis(sim_g, jnp.asarray(_COLMAP_GRAPH), axis=1)
    labels_graph = jnp.zeros((_G,), jnp.int32)
    return logits_node, labels_node, logits_graph, labels_graph
